# Initial kernel scaffold; baseline (speedup 1.0000x reference)
import jax
import jax.numpy as jnp
from jax.experimental import pallas as pl


def kernel(x, edge_index, Wl0, bl0, Wr0, g0, be0, Wl1, bl1, Wr1, g1, be1, Wl2, bl2, Wr2, g2, be2, Wl3, bl3, Wr3, g3, be3, Wl4, bl4, Wr4, g4, be4, cW1, cb1, cW2, cb2):
    raise NotImplementedError("write your pallas kernel here")


# speedup vs baseline: 4.6752x; 4.6752x over previous
import functools

import jax
import jax.numpy as jnp
from jax import lax
from jax.experimental import pallas as pl
from jax.experimental.pallas import tpu as pltpu
from jax.experimental.pallas import tpu_sc as plsc


def _make_agg(n, d, nw, nch, k, n_acc):
  info = plsc.get_sparse_core_info()
  nc, ns = info.num_cores, info.num_subcores
  mesh = plsc.VectorSubcoreMesh(core_axis_name="c", subcore_axis_name="s")
  g8 = 8
  ng = nch // g8
  scratch = [
      pltpu.VMEM((g8, k), jnp.int32),
      pltpu.VMEM((g8, k), jnp.int32),
      pltpu.VMEM((k, d), jnp.float32),
      pltpu.VMEM_SHARED((n_acc, d), jnp.float32),
      pltpu.SemaphoreType.DMA,
  ]

  def body(x_hbm, src_hbm, dst_hbm, zacc_hbm, s_out,
           srcg, dstg, rows0, acc_sh, sem_r0):
    c = lax.axis_index("c")
    s = lax.axis_index("s")
    wid = s * nc + c
    rz = n_acc // ns
    pltpu.sync_copy(zacc_hbm.at[pl.ds(s * rz, rz)],
                    acc_sh.at[pl.ds(s * rz, rz)])
    plsc.subcore_barrier()

    def group_body(g, carry):
      off = pl.multiple_of(g * g8, 8)
      pltpu.sync_copy(src_hbm.at[wid, pl.ds(off, g8)], srcg)
      pltpu.sync_copy(dst_hbm.at[wid, pl.ds(off, g8)], dstg)
      for t in range(g8):
        pltpu.async_copy(x_hbm.at[srcg.at[t]], rows0, sem_r0).wait()
        pltpu.sync_copy(rows0, acc_sh.at[dstg.at[t]], add=True)
      return carry

    lax.fori_loop(0, ng, group_body, 0)
    plsc.subcore_barrier()
    pltpu.sync_copy(acc_sh.at[pl.ds(s * rz, rz)],
                    s_out.at[c, pl.ds(s * rz, rz)])

  return pl.kernel(
      body, mesh=mesh,
      out_type=jax.ShapeDtypeStruct((nc, n_acc, d), jnp.float32),
      scratch_types=scratch)


def _dense(layer, n, dout):
  def body(*refs):
    if layer == 0:
      (s_ref, cnt_ref, x_ref, wl_ref, bl_ref, wr_ref, g_ref, be_ref,
       o_ref, inv_ref) = refs
      cnt = cnt_ref[...]
      inv = 1.0 / jnp.maximum(cnt[0, :n, 0:1] + cnt[1, :n, 0:1], 1.0)
      inv_ref[...] = inv
    elif layer == 4:
      (s_ref, invin_ref, x_ref, wl_ref, bl_ref, wr_ref, g_ref, be_ref,
       cw1_ref, cb1_ref, cw2_ref, cb2_ref, o_ref) = refs
      inv = invin_ref[...]
    else:
      (s_ref, invin_ref, x_ref, wl_ref, bl_ref, wr_ref, g_ref, be_ref,
       o_ref) = refs
      inv = invin_ref[...]
    sp = s_ref[...]
    mean = (sp[0, :n] + sp[1, :n]) * inv
    out = (jnp.dot(mean, wl_ref[...], preferred_element_type=jnp.float32)
           + bl_ref[...]
           + jnp.dot(x_ref[...], wr_ref[...],
                     preferred_element_type=jnp.float32))
    nrm = jnp.sqrt(jnp.sum(out * out, axis=1, keepdims=True))
    out = out / jnp.maximum(nrm, 1e-12)
    mu = jnp.mean(out, axis=0, keepdims=True)
    var = jnp.mean((out - mu) ** 2, axis=0, keepdims=True)
    out = (out - mu) / jnp.sqrt(var + 1e-5) * g_ref[...] + be_ref[...]
    if layer < 4:
      o_ref[...] = jnp.maximum(out, 0.0)
    else:
      h = jnp.maximum(
          jnp.dot(out, cw1_ref[...], preferred_element_type=jnp.float32)
          + cb1_ref[...], 0.0)
      o_ref[...] = (jnp.dot(h, cw2_ref[...],
                            preferred_element_type=jnp.float32)
                    + cb2_ref[...])

  if layer == 0:
    out_shape = (jax.ShapeDtypeStruct((n, dout), jnp.float32),
                 jax.ShapeDtypeStruct((n, 1), jnp.float32))
  elif layer == 4:
    out_shape = jax.ShapeDtypeStruct((n, 1), jnp.float32)
  else:
    out_shape = jax.ShapeDtypeStruct((n, dout), jnp.float32)
  return pl.pallas_call(body, out_shape=out_shape)


def kernel(x, edge_index,
           Wl0, bl0, Wr0, g0, be0,
           Wl1, bl1, Wr1, g1, be1,
           Wl2, bl2, Wr2, g2, be2,
           Wl3, bl3, Wr3, g3, be3,
           Wl4, bl4, Wr4, g4, be4,
           cW1, cb1, cW2, cb2):
  n, d = x.shape
  e = edge_index.shape[1]
  info = plsc.get_sparse_core_info()
  nw = info.num_cores * info.num_subcores
  ns = info.num_subcores
  k = 64
  nch = -(-e // (nw * k * 16)) * 16
  e_pad = nw * nch * k
  pad = e_pad - e
  n_acc = -(-n // (ns * 8)) * ns * 8
  npad_rows = n_acc - n
  src = edge_index[0]
  dst = edge_index[1]
  if pad:
    ar = jnp.arange(pad, dtype=jnp.int32)
    src = jnp.concatenate([src, ar % n])
    dst = jnp.concatenate([dst, n + (ar % max(npad_rows, 1))])
  src_p = src.reshape(nw, nch, k)
  dst_p = dst.reshape(nw, nch, k)
  zacc = jnp.zeros((n_acc, d), jnp.float32)
  ones_tbl = jnp.ones((n, d), jnp.float32)

  agg = _make_agg(n, d, nw, nch, k, n_acc)

  layer_params = [
      (Wl0, bl0, Wr0, g0, be0),
      (Wl1, bl1, Wr1, g1, be1),
      (Wl2, bl2, Wr2, g2, be2),
      (Wl3, bl3, Wr3, g3, be3),
      (Wl4, bl4, Wr4, g4, be4),
  ]

  cnt_parts = agg(ones_tbl, src_p, dst_p, zacc)
  xi = x
  inv = None
  logits = None
  for i, (wl, bl, wr, g, be) in enumerate(layer_params):
    bl2 = bl.reshape(1, -1)
    g2 = g.reshape(1, -1)
    be2 = be.reshape(1, -1)
    dout = wl.shape[1]
    s_parts = agg(xi, src_p, dst_p, zacc)
    if i == 0:
      xi, inv = _dense(0, n, dout)(s_parts, cnt_parts, xi, wl, bl2, wr,
                                   g2, be2)
    elif i < 4:
      xi = _dense(i, n, dout)(s_parts, inv, xi, wl, bl2, wr, g2, be2)
    else:
      logits = _dense(4, n, dout)(s_parts, inv, xi, wl, bl2, wr, g2, be2,
                                  cW1, cb1.reshape(1, -1),
                                  cW2, cb2.reshape(1, -1))
  return logits[:, 0]


# speedup vs baseline: 8.1580x; 1.7449x over previous
import functools

import jax
import jax.numpy as jnp
from jax import lax
from jax.experimental import pallas as pl
from jax.experimental.pallas import tpu as pltpu
from jax.experimental.pallas import tpu_sc as plsc


def _make_agg(n, d, nw, nch, k, n_acc):
  info = plsc.get_sparse_core_info()
  nc, ns = info.num_cores, info.num_subcores
  mesh = plsc.VectorSubcoreMesh(core_axis_name="c", subcore_axis_name="s")
  g8 = 8
  ng = nch // g8
  scratch = [
      pltpu.VMEM((g8, k), jnp.int32),
      pltpu.VMEM((g8, k), jnp.int32),
      pltpu.VMEM((g8, k), jnp.int32),
      pltpu.VMEM((g8, k), jnp.int32),
      pltpu.VMEM((k, d), jnp.float32),
      pltpu.VMEM((k, d), jnp.float32),
      pltpu.VMEM_SHARED((n_acc, d), jnp.float32),
      pltpu.SemaphoreType.DMA,
      pltpu.SemaphoreType.DMA,
      pltpu.SemaphoreType.DMA,
      pltpu.SemaphoreType.DMA,
  ]

  def body(x_hbm, src_hbm, dst_hbm, zacc_hbm, s_out,
           srcg0, srcg1, dstg0, dstg1, rows0, rows1, acc_sh,
           sem_i0, sem_i1, sem_r0, sem_r1):
    c = lax.axis_index("c")
    s = lax.axis_index("s")
    wid = s * nc + c
    srcg = (srcg0, srcg1)
    dstg = (dstg0, dstg1)
    sem_i = (sem_i0, sem_i1)
    rows = (rows0, rows1)
    sem_r = (sem_r0, sem_r1)
    rz = n_acc // ns
    pltpu.sync_copy(zacc_hbm.at[pl.ds(s * rz, rz)],
                    acc_sh.at[pl.ds(s * rz, rz)])
    plsc.subcore_barrier()

    def start_idx(g, gs):
      return (
          pltpu.async_copy(src_hbm.at[wid, pl.ds(g * g8, g8)], srcg[gs],
                           sem_i[gs]),
          pltpu.async_copy(dst_hbm.at[wid, pl.ds(g * g8, g8)], dstg[gs],
                           sem_i[gs]),
      )

    def start_gather(gs, t, b):
      return pltpu.async_copy(x_hbm.at[srcg[gs].at[t]], rows[b], sem_r[b])

    hi = [start_idx(0, 0), start_idx(1, 1)]
    for h in hi[0]:
      h.wait()
    hg = [start_gather(0, 0, 0), start_gather(0, 1, 1)]
    for g in range(ng):
      gs = g % 2
      if g + 1 < ng:
        for h in hi[gs ^ 1]:
          h.wait()
      for t in range(g8):
        b = t % 2
        j = g * g8 + t
        hg[b].wait()
        pltpu.sync_copy(rows[b], acc_sh.at[dstg[gs].at[t]], add=True)
        nxt = j + 2
        if nxt < nch:
          hg[b] = start_gather((nxt // g8) % 2, nxt % g8, b)
      if g + 2 < ng:
        hi[gs] = start_idx(g + 2, gs)
    plsc.subcore_barrier()
    pltpu.sync_copy(acc_sh.at[pl.ds(s * rz, rz)],
                    s_out.at[c, pl.ds(s * rz, rz)])

  return pl.kernel(
      body, mesh=mesh,
      out_type=jax.ShapeDtypeStruct((nc, n_acc, d), jnp.float32),
      scratch_types=scratch)


def _dense(layer, n, dout):
  def body(*refs):
    if layer == 0:
      (s_ref, cnt_ref, x_ref, wl_ref, bl_ref, wr_ref, g_ref, be_ref,
       o_ref, inv_ref) = refs
      cnt = cnt_ref[...]
      inv = 1.0 / jnp.maximum(cnt[0, :n, 0:1] + cnt[1, :n, 0:1], 1.0)
      inv_ref[...] = inv
    elif layer == 4:
      (s_ref, invin_ref, x_ref, wl_ref, bl_ref, wr_ref, g_ref, be_ref,
       cw1_ref, cb1_ref, cw2_ref, cb2_ref, o_ref) = refs
      inv = invin_ref[...]
    else:
      (s_ref, invin_ref, x_ref, wl_ref, bl_ref, wr_ref, g_ref, be_ref,
       o_ref) = refs
      inv = invin_ref[...]
    sp = s_ref[...]
    mean = (sp[0, :n] + sp[1, :n]) * inv
    out = (jnp.dot(mean, wl_ref[...], preferred_element_type=jnp.float32)
           + bl_ref[...]
           + jnp.dot(x_ref[...], wr_ref[...],
                     preferred_element_type=jnp.float32))
    nrm = jnp.sqrt(jnp.sum(out * out, axis=1, keepdims=True))
    out = out / jnp.maximum(nrm, 1e-12)
    mu = jnp.mean(out, axis=0, keepdims=True)
    var = jnp.mean((out - mu) ** 2, axis=0, keepdims=True)
    out = (out - mu) / jnp.sqrt(var + 1e-5) * g_ref[...] + be_ref[...]
    if layer < 4:
      o_ref[...] = jnp.maximum(out, 0.0)
    else:
      h = jnp.maximum(
          jnp.dot(out, cw1_ref[...], preferred_element_type=jnp.float32)
          + cb1_ref[...], 0.0)
      o_ref[...] = (jnp.dot(h, cw2_ref[...],
                            preferred_element_type=jnp.float32)
                    + cb2_ref[...])

  if layer == 0:
    out_shape = (jax.ShapeDtypeStruct((n, dout), jnp.float32),
                 jax.ShapeDtypeStruct((n, 1), jnp.float32))
  elif layer == 4:
    out_shape = jax.ShapeDtypeStruct((n, 1), jnp.float32)
  else:
    out_shape = jax.ShapeDtypeStruct((n, dout), jnp.float32)
  return pl.pallas_call(body, out_shape=out_shape)


def kernel(x, edge_index,
           Wl0, bl0, Wr0, g0, be0,
           Wl1, bl1, Wr1, g1, be1,
           Wl2, bl2, Wr2, g2, be2,
           Wl3, bl3, Wr3, g3, be3,
           Wl4, bl4, Wr4, g4, be4,
           cW1, cb1, cW2, cb2):
  n, d = x.shape
  e = edge_index.shape[1]
  info = plsc.get_sparse_core_info()
  nw = info.num_cores * info.num_subcores
  ns = info.num_subcores
  k = 64
  nch = -(-e // (nw * k * 16)) * 16
  e_pad = nw * nch * k
  pad = e_pad - e
  n_acc = -(-n // (ns * 8)) * ns * 8
  npad_rows = n_acc - n
  src = edge_index[0]
  dst = edge_index[1]
  if pad:
    ar = jnp.arange(pad, dtype=jnp.int32)
    src = jnp.concatenate([src, ar % n])
    dst = jnp.concatenate([dst, n + (ar % max(npad_rows, 1))])
  src_p = src.reshape(nw, nch, k)
  dst_p = dst.reshape(nw, nch, k)
  zacc = jnp.zeros((n_acc, d), jnp.float32)
  ones_tbl = jnp.ones((n, d), jnp.float32)

  agg = _make_agg(n, d, nw, nch, k, n_acc)

  layer_params = [
      (Wl0, bl0, Wr0, g0, be0),
      (Wl1, bl1, Wr1, g1, be1),
      (Wl2, bl2, Wr2, g2, be2),
      (Wl3, bl3, Wr3, g3, be3),
      (Wl4, bl4, Wr4, g4, be4),
  ]

  cnt_parts = agg(ones_tbl, src_p, dst_p, zacc)
  xi = x
  inv = None
  logits = None
  for i, (wl, bl, wr, g, be) in enumerate(layer_params):
    bl2 = bl.reshape(1, -1)
    g2 = g.reshape(1, -1)
    be2 = be.reshape(1, -1)
    dout = wl.shape[1]
    s_parts = agg(xi, src_p, dst_p, zacc)
    if i == 0:
      xi, inv = _dense(0, n, dout)(s_parts, cnt_parts, xi, wl, bl2, wr,
                                   g2, be2)
    elif i < 4:
      xi = _dense(i, n, dout)(s_parts, inv, xi, wl, bl2, wr, g2, be2)
    else:
      logits = _dense(4, n, dout)(s_parts, inv, xi, wl, bl2, wr, g2, be2,
                                  cW1, cb1.reshape(1, -1),
                                  cW2, cb2.reshape(1, -1))
  return logits[:, 0]


# speedup vs baseline: 9.8679x; 1.2096x over previous
import functools

import jax
import jax.numpy as jnp
from jax import lax
from jax.experimental import pallas as pl
from jax.experimental.pallas import tpu as pltpu
from jax.experimental.pallas import tpu_sc as plsc


def _make_agg(n, d, nw, nch, k, n_acc):
  info = plsc.get_sparse_core_info()
  nc, ns = info.num_cores, info.num_subcores
  mesh = plsc.VectorSubcoreMesh(core_axis_name="c", subcore_axis_name="s")
  g8 = 8
  ng = nch // g8
  scratch = [
      pltpu.VMEM((g8, k), jnp.int32),
      pltpu.VMEM((g8, k), jnp.int32),
      pltpu.VMEM((g8, k), jnp.int32),
      pltpu.VMEM((g8, k), jnp.int32),
      pltpu.VMEM((k, d), jnp.float32),
      pltpu.VMEM((k, d), jnp.float32),
      pltpu.VMEM((k, d), jnp.float32),
      pltpu.VMEM((k, d), jnp.float32),
      pltpu.VMEM_SHARED((n_acc, d), jnp.float32),
      pltpu.SemaphoreType.DMA,
      pltpu.SemaphoreType.DMA,
      pltpu.SemaphoreType.DMA,
      pltpu.SemaphoreType.DMA,
      pltpu.SemaphoreType.DMA,
      pltpu.SemaphoreType.DMA,
      pltpu.SemaphoreType.DMA,
      pltpu.SemaphoreType.DMA,
      pltpu.SemaphoreType.DMA,
      pltpu.SemaphoreType.DMA,
  ]

  def body(x_hbm, src_hbm, dst_hbm, zacc_hbm, s_out,
           srcg0, srcg1, dstg0, dstg1, rows0, rows1, rows2, rows3, acc_sh,
           sem_i0, sem_i1, sem_r0, sem_r1, sem_r2, sem_r3,
           sem_s0, sem_s1, sem_s2, sem_s3):
    c = lax.axis_index("c")
    s = lax.axis_index("s")
    wid = s * nc + c
    srcg = (srcg0, srcg1)
    dstg = (dstg0, dstg1)
    sem_i = (sem_i0, sem_i1)
    rows = (rows0, rows1, rows2, rows3)
    sem_r = (sem_r0, sem_r1, sem_r2, sem_r3)
    sem_s = (sem_s0, sem_s1, sem_s2, sem_s3)
    rz = n_acc // ns
    pltpu.sync_copy(zacc_hbm.at[pl.ds(s * rz, rz)],
                    acc_sh.at[pl.ds(s * rz, rz)])
    plsc.subcore_barrier()

    def start_idx(g, gs):
      return (
          pltpu.async_copy(src_hbm.at[wid, pl.ds(g * g8, g8)], srcg[gs],
                           sem_i[gs]),
          pltpu.async_copy(dst_hbm.at[wid, pl.ds(g * g8, g8)], dstg[gs],
                           sem_i[gs]),
      )

    def start_gather(j, b):
      return pltpu.async_copy(x_hbm.at[srcg[(j // g8) % 2].at[j % g8]],
                              rows[b], sem_r[b])

    def start_scatter(j, b):
      return pltpu.async_copy(rows[b], acc_sh.at[dstg[(j // g8) % 2].at[j % g8]],
                              sem_s[b], add=True)

    hi = [start_idx(0, 0), start_idx(1, 1)]
    for h in hi[0]:
      h.wait()
    hg = [start_gather(0, 0), start_gather(1, 1), start_gather(2, 2), None]
    hs = [None, None, None, None]
    for j in range(nch):
      g, t, gs, b = j // g8, j % g8, (j // g8) % 2, j % 4
      if t == 5 and g + 1 < ng:
        for h in hi[gs ^ 1]:
          h.wait()
      hg[b].wait()
      hs[b] = start_scatter(j, b)
      nxt = j + 3
      if nxt < nch:
        bb = nxt % 4
        if hs[bb] is not None:
          hs[bb].wait()
        hg[bb] = start_gather(nxt, bb)
      if t == 7 and g + 2 < ng:
        hi[gs] = start_idx(g + 2, gs)
    for b in range(4):
      if hs[b] is not None:
        hs[b].wait()
    plsc.subcore_barrier()
    pltpu.sync_copy(acc_sh.at[pl.ds(s * rz, rz)],
                    s_out.at[c, pl.ds(s * rz, rz)])

  return pl.kernel(
      body, mesh=mesh,
      out_type=jax.ShapeDtypeStruct((nc, n_acc, d), jnp.float32),
      scratch_types=scratch)


def _dense(layer, n, dout):
  def body(*refs):
    if layer == 0:
      (s_ref, cnt_ref, x_ref, wl_ref, bl_ref, wr_ref, g_ref, be_ref,
       o_ref, inv_ref) = refs
      cnt = cnt_ref[...]
      inv = 1.0 / jnp.maximum(cnt[0, :n, 0:1] + cnt[1, :n, 0:1], 1.0)
      inv_ref[...] = inv
    elif layer == 4:
      (s_ref, invin_ref, x_ref, wl_ref, bl_ref, wr_ref, g_ref, be_ref,
       cw1_ref, cb1_ref, cw2_ref, cb2_ref, o_ref) = refs
      inv = invin_ref[...]
    else:
      (s_ref, invin_ref, x_ref, wl_ref, bl_ref, wr_ref, g_ref, be_ref,
       o_ref) = refs
      inv = invin_ref[...]
    sp = s_ref[...]
    mean = (sp[0, :n] + sp[1, :n]) * inv
    out = (jnp.dot(mean, wl_ref[...], preferred_element_type=jnp.float32)
           + bl_ref[...]
           + jnp.dot(x_ref[...], wr_ref[...],
                     preferred_element_type=jnp.float32))
    nrm = jnp.sqrt(jnp.sum(out * out, axis=1, keepdims=True))
    out = out / jnp.maximum(nrm, 1e-12)
    mu = jnp.mean(out, axis=0, keepdims=True)
    var = jnp.mean((out - mu) ** 2, axis=0, keepdims=True)
    out = (out - mu) / jnp.sqrt(var + 1e-5) * g_ref[...] + be_ref[...]
    if layer < 4:
      o_ref[...] = jnp.maximum(out, 0.0)
    else:
      h = jnp.maximum(
          jnp.dot(out, cw1_ref[...], preferred_element_type=jnp.float32)
          + cb1_ref[...], 0.0)
      o_ref[...] = (jnp.dot(h, cw2_ref[...],
                            preferred_element_type=jnp.float32)
                    + cb2_ref[...])

  if layer == 0:
    out_shape = (jax.ShapeDtypeStruct((n, dout), jnp.float32),
                 jax.ShapeDtypeStruct((n, 1), jnp.float32))
  elif layer == 4:
    out_shape = jax.ShapeDtypeStruct((n, 1), jnp.float32)
  else:
    out_shape = jax.ShapeDtypeStruct((n, dout), jnp.float32)
  return pl.pallas_call(body, out_shape=out_shape)


def kernel(x, edge_index,
           Wl0, bl0, Wr0, g0, be0,
           Wl1, bl1, Wr1, g1, be1,
           Wl2, bl2, Wr2, g2, be2,
           Wl3, bl3, Wr3, g3, be3,
           Wl4, bl4, Wr4, g4, be4,
           cW1, cb1, cW2, cb2):
  n, d = x.shape
  e = edge_index.shape[1]
  info = plsc.get_sparse_core_info()
  nw = info.num_cores * info.num_subcores
  ns = info.num_subcores
  k = 64
  nch = -(-e // (nw * k * 16)) * 16
  e_pad = nw * nch * k
  pad = e_pad - e
  n_acc = -(-n // (ns * 8)) * ns * 8
  npad_rows = n_acc - n
  src = edge_index[0]
  dst = edge_index[1]
  if pad:
    ar = jnp.arange(pad, dtype=jnp.int32)
    src = jnp.concatenate([src, ar % n])
    dst = jnp.concatenate([dst, n + (ar % max(npad_rows, 1))])
  src_p = src.reshape(nw, nch, k)
  dst_p = dst.reshape(nw, nch, k)
  zacc = jnp.zeros((n_acc, d), jnp.float32)
  ones_tbl = jnp.ones((n, d), jnp.float32)

  agg = _make_agg(n, d, nw, nch, k, n_acc)

  layer_params = [
      (Wl0, bl0, Wr0, g0, be0),
      (Wl1, bl1, Wr1, g1, be1),
      (Wl2, bl2, Wr2, g2, be2),
      (Wl3, bl3, Wr3, g3, be3),
      (Wl4, bl4, Wr4, g4, be4),
  ]

  cnt_parts = agg(ones_tbl, src_p, dst_p, zacc)
  xi = x
  inv = None
  logits = None
  for i, (wl, bl, wr, g, be) in enumerate(layer_params):
    bl2 = bl.reshape(1, -1)
    g2 = g.reshape(1, -1)
    be2 = be.reshape(1, -1)
    dout = wl.shape[1]
    s_parts = agg(xi, src_p, dst_p, zacc)
    if i == 0:
      xi, inv = _dense(0, n, dout)(s_parts, cnt_parts, xi, wl, bl2, wr,
                                   g2, be2)
    elif i < 4:
      xi = _dense(i, n, dout)(s_parts, inv, xi, wl, bl2, wr, g2, be2)
    else:
      logits = _dense(4, n, dout)(s_parts, inv, xi, wl, bl2, wr, g2, be2,
                                  cW1, cb1.reshape(1, -1),
                                  cW2, cb2.reshape(1, -1))
  return logits[:, 0]


# speedup vs baseline: 10.2205x; 1.0357x over previous
import functools

import jax
import jax.numpy as jnp
from jax import lax
from jax.experimental import pallas as pl
from jax.experimental.pallas import tpu as pltpu
from jax.experimental.pallas import tpu_sc as plsc


def _make_agg(n, d, nw, nch, k, n_acc):
  info = plsc.get_sparse_core_info()
  nc, ns = info.num_cores, info.num_subcores
  mesh = plsc.VectorSubcoreMesh(core_axis_name="c", subcore_axis_name="s")
  g8 = 8
  ng = nch // g8
  scratch = [
      pltpu.VMEM((g8, k), jnp.int32),
      pltpu.VMEM((g8, k), jnp.int32),
      pltpu.VMEM((g8, k), jnp.int32),
      pltpu.VMEM((g8, k), jnp.int32),
      pltpu.VMEM((k, d), jnp.float32),
      pltpu.VMEM((k, d), jnp.float32),
      pltpu.VMEM((k, d), jnp.float32),
      pltpu.VMEM((k, d), jnp.float32),
      pltpu.VMEM_SHARED((n_acc, d), jnp.float32),
      pltpu.SemaphoreType.DMA,
      pltpu.SemaphoreType.DMA,
      pltpu.SemaphoreType.DMA,
      pltpu.SemaphoreType.DMA,
      pltpu.SemaphoreType.DMA,
      pltpu.SemaphoreType.DMA,
      pltpu.SemaphoreType.DMA,
      pltpu.SemaphoreType.DMA,
      pltpu.SemaphoreType.DMA,
      pltpu.SemaphoreType.DMA,
  ]

  def body(x_hbm, src_hbm, dst_hbm, zacc_hbm, s_out,
           srcg0, srcg1, dstg0, dstg1, rows0, rows1, rows2, rows3, acc_sh,
           sem_i0, sem_i1, sem_r0, sem_r1, sem_r2, sem_r3,
           sem_s0, sem_s1, sem_s2, sem_s3):
    c = lax.axis_index("c")
    s = lax.axis_index("s")
    wid = s * nc + c
    srcg = (srcg0, srcg1)
    dstg = (dstg0, dstg1)
    sem_i = (sem_i0, sem_i1)
    rows = (rows0, rows1, rows2, rows3)
    sem_r = (sem_r0, sem_r1, sem_r2, sem_r3)
    sem_s = (sem_s0, sem_s1, sem_s2, sem_s3)
    rz = n_acc // ns
    pltpu.sync_copy(zacc_hbm.at[pl.ds(s * rz, rz)],
                    acc_sh.at[pl.ds(s * rz, rz)])
    plsc.subcore_barrier()

    def start_idx(g, gs):
      return (
          pltpu.async_copy(src_hbm.at[wid, pl.ds(g * g8, g8)], srcg[gs],
                           sem_i[gs]),
          pltpu.async_copy(dst_hbm.at[wid, pl.ds(g * g8, g8)], dstg[gs],
                           sem_i[gs]),
      )

    def start_gather(j, b):
      return pltpu.async_copy(x_hbm.at[srcg[(j // g8) % 2].at[j % g8]],
                              rows[b], sem_r[b])

    def start_scatter(j, b):
      return pltpu.async_copy(rows[b], acc_sh.at[dstg[(j // g8) % 2].at[j % g8]],
                              sem_s[b], add=True)

    hi = [start_idx(0, 0), start_idx(1, 1)]
    for h in hi[0]:
      h.wait()
    hg = [start_gather(0, 0), start_gather(1, 1), start_gather(2, 2), None]
    hs = [None, None, None, None]
    for j in range(nch):
      g, t, gs, b = j // g8, j % g8, (j // g8) % 2, j % 4
      if t == 5 and g + 1 < ng:
        for h in hi[gs ^ 1]:
          h.wait()
      hg[b].wait()
      hs[b] = start_scatter(j, b)
      nxt = j + 3
      if nxt < nch:
        bb = nxt % 4
        if hs[bb] is not None:
          hs[bb].wait()
        hg[bb] = start_gather(nxt, bb)
      if t == 7 and g + 2 < ng:
        hi[gs] = start_idx(g + 2, gs)
    for b in range(4):
      if hs[b] is not None:
        hs[b].wait()
    plsc.subcore_barrier()
    pltpu.sync_copy(acc_sh.at[pl.ds(s * rz, rz)],
                    s_out.at[c, pl.ds(s * rz, rz)])

  return pl.kernel(
      body, mesh=mesh,
      out_type=jax.ShapeDtypeStruct((nc, n_acc, d), jnp.float32),
      scratch_types=scratch)


def _make_cnt(n, d, nw, nch, k, n_acc):
  info = plsc.get_sparse_core_info()
  nc, ns = info.num_cores, info.num_subcores
  mesh = plsc.VectorSubcoreMesh(core_axis_name="c", subcore_axis_name="s")
  g8 = 8
  ng = nch // g8
  scratch = [
      pltpu.VMEM((g8, k), jnp.int32),
      pltpu.VMEM((g8, k), jnp.int32),
      pltpu.VMEM((k, d), jnp.float32),
      pltpu.VMEM_SHARED((n_acc, d), jnp.float32),
      pltpu.SemaphoreType.DMA,
      pltpu.SemaphoreType.DMA,
      pltpu.SemaphoreType.DMA,
      pltpu.SemaphoreType.DMA,
  ]

  def body(ones_hbm, dst_hbm, zacc_hbm, cnt_out,
           dstg0, dstg1, onesv, acc_sh, sem_i0, sem_i1, sem_s0, sem_s1):
    c = lax.axis_index("c")
    s = lax.axis_index("s")
    wid = s * nc + c
    dstg = (dstg0, dstg1)
    sem_i = (sem_i0, sem_i1)
    sem_s = (sem_s0, sem_s1)
    rz = n_acc // ns
    pltpu.sync_copy(zacc_hbm.at[pl.ds(s * rz, rz)],
                    acc_sh.at[pl.ds(s * rz, rz)])
    pltpu.sync_copy(ones_hbm, onesv)
    plsc.subcore_barrier()

    def start_idx(g, gs):
      return pltpu.async_copy(dst_hbm.at[wid, pl.ds(g * g8, g8)], dstg[gs],
                              sem_i[gs])

    hi = [start_idx(0, 0), start_idx(1, 1)]
    hi[0].wait()
    hs = []
    for g in range(ng):
      gs = g % 2
      if g + 1 < ng:
        hi[gs ^ 1].wait()
      for t in range(g8):
        hs.append(pltpu.async_copy(onesv, acc_sh.at[dstg[gs].at[t]],
                                   sem_s[gs], add=True))
      for h in hs:
        h.wait()
      hs = []
      if g + 2 < ng:
        hi[gs] = start_idx(g + 2, gs)
    plsc.subcore_barrier()
    pltpu.sync_copy(acc_sh.at[pl.ds(s * rz, rz)],
                    cnt_out.at[c, pl.ds(s * rz, rz)])

  return pl.kernel(
      body, mesh=mesh,
      out_type=jax.ShapeDtypeStruct((nc, n_acc, d), jnp.float32),
      scratch_types=scratch)


def _dense(layer, n, dout):
  def body(*refs):
    if layer == 0:
      (s_ref, cnt_ref, x_ref, wl_ref, bl_ref, wr_ref, g_ref, be_ref,
       o_ref, inv_ref) = refs
      cnt = cnt_ref[...]
      inv = 1.0 / jnp.maximum(cnt[0, :n, 0:1] + cnt[1, :n, 0:1], 1.0)
      inv_ref[...] = inv
    elif layer == 4:
      (s_ref, invin_ref, x_ref, wl_ref, bl_ref, wr_ref, g_ref, be_ref,
       cw1_ref, cb1_ref, cw2_ref, cb2_ref, o_ref) = refs
      inv = invin_ref[...]
    else:
      (s_ref, invin_ref, x_ref, wl_ref, bl_ref, wr_ref, g_ref, be_ref,
       o_ref) = refs
      inv = invin_ref[...]
    sp = s_ref[...]
    mean = (sp[0, :n] + sp[1, :n]) * inv
    out = (jnp.dot(mean, wl_ref[...], preferred_element_type=jnp.float32)
           + bl_ref[...]
           + jnp.dot(x_ref[...], wr_ref[...],
                     preferred_element_type=jnp.float32))
    nrm = jnp.sqrt(jnp.sum(out * out, axis=1, keepdims=True))
    out = out / jnp.maximum(nrm, 1e-12)
    mu = jnp.mean(out, axis=0, keepdims=True)
    var = jnp.mean((out - mu) ** 2, axis=0, keepdims=True)
    out = (out - mu) / jnp.sqrt(var + 1e-5) * g_ref[...] + be_ref[...]
    if layer < 4:
      o_ref[...] = jnp.maximum(out, 0.0)
    else:
      h = jnp.maximum(
          jnp.dot(out, cw1_ref[...], preferred_element_type=jnp.float32)
          + cb1_ref[...], 0.0)
      o_ref[...] = (jnp.dot(h, cw2_ref[...],
                            preferred_element_type=jnp.float32)
                    + cb2_ref[...])

  if layer == 0:
    out_shape = (jax.ShapeDtypeStruct((n, dout), jnp.float32),
                 jax.ShapeDtypeStruct((n, 1), jnp.float32))
  elif layer == 4:
    out_shape = jax.ShapeDtypeStruct((n, 1), jnp.float32)
  else:
    out_shape = jax.ShapeDtypeStruct((n, dout), jnp.float32)
  return pl.pallas_call(body, out_shape=out_shape)


def kernel(x, edge_index,
           Wl0, bl0, Wr0, g0, be0,
           Wl1, bl1, Wr1, g1, be1,
           Wl2, bl2, Wr2, g2, be2,
           Wl3, bl3, Wr3, g3, be3,
           Wl4, bl4, Wr4, g4, be4,
           cW1, cb1, cW2, cb2):
  n, d = x.shape
  e = edge_index.shape[1]
  info = plsc.get_sparse_core_info()
  nw = info.num_cores * info.num_subcores
  ns = info.num_subcores
  k = 64
  nch = -(-e // (nw * k * 16)) * 16
  e_pad = nw * nch * k
  pad = e_pad - e
  n_acc = -(-n // (ns * 8)) * ns * 8
  npad_rows = n_acc - n
  src = edge_index[0]
  dst = edge_index[1]
  if pad:
    ar = jnp.arange(pad, dtype=jnp.int32)
    src = jnp.concatenate([src, ar % n])
    dst = jnp.concatenate([dst, n + (ar % max(npad_rows, 1))])
  src_p = src.reshape(nw, nch, k)
  dst_p = dst.reshape(nw, nch, k)
  zacc = jnp.zeros((n_acc, d), jnp.float32)
  ones_rows = jnp.ones((k, d), jnp.float32)

  agg = _make_agg(n, d, nw, nch, k, n_acc)
  cntk = _make_cnt(n, d, nw, nch, k, n_acc)

  layer_params = [
      (Wl0, bl0, Wr0, g0, be0),
      (Wl1, bl1, Wr1, g1, be1),
      (Wl2, bl2, Wr2, g2, be2),
      (Wl3, bl3, Wr3, g3, be3),
      (Wl4, bl4, Wr4, g4, be4),
  ]

  cnt_parts = cntk(ones_rows, dst_p, zacc)
  xi = x
  inv = None
  logits = None
  for i, (wl, bl, wr, g, be) in enumerate(layer_params):
    bl2 = bl.reshape(1, -1)
    g2 = g.reshape(1, -1)
    be2 = be.reshape(1, -1)
    dout = wl.shape[1]
    s_parts = agg(xi, src_p, dst_p, zacc)
    if i == 0:
      xi, inv = _dense(0, n, dout)(s_parts, cnt_parts, xi, wl, bl2, wr,
                                   g2, be2)
    elif i < 4:
      xi = _dense(i, n, dout)(s_parts, inv, xi, wl, bl2, wr, g2, be2)
    else:
      logits = _dense(4, n, dout)(s_parts, inv, xi, wl, bl2, wr, g2, be2,
                                  cW1, cb1.reshape(1, -1),
                                  cW2, cb2.reshape(1, -1))
  return logits[:, 0]


# speedup vs baseline: 10.5941x; 1.0366x over previous
import functools

import jax
import jax.numpy as jnp
from jax import lax
from jax.experimental import pallas as pl
from jax.experimental.pallas import tpu as pltpu
from jax.experimental.pallas import tpu_sc as plsc


def _make_agg(n, d, nw, nch, k, n_acc):
  info = plsc.get_sparse_core_info()
  nc, ns = info.num_cores, info.num_subcores
  mesh = plsc.VectorSubcoreMesh(core_axis_name="c", subcore_axis_name="s")
  g8 = 8
  ng = nch // g8
  scratch = [
      pltpu.VMEM((g8, k), jnp.int32),
      pltpu.VMEM((g8, k), jnp.int32),
      pltpu.VMEM((g8, k), jnp.int32),
      pltpu.VMEM((g8, k), jnp.int32),
      pltpu.VMEM((k, d), jnp.float32),
      pltpu.VMEM((k, d), jnp.float32),
      pltpu.VMEM((k, d), jnp.float32),
      pltpu.VMEM_SHARED((n_acc, d), jnp.float32),
      pltpu.SemaphoreType.DMA,
      pltpu.SemaphoreType.DMA,
      pltpu.SemaphoreType.DMA,
      pltpu.SemaphoreType.DMA,
      pltpu.SemaphoreType.DMA,
      pltpu.SemaphoreType.DMA,
      pltpu.SemaphoreType.DMA,
      pltpu.SemaphoreType.DMA,
  ]
  nb = 3

  def body(x_hbm, src_hbm, dst_hbm, zacc_hbm, s_out,
           srcg0, srcg1, dstg0, dstg1, rows0, rows1, rows2, acc_sh,
           sem_i0, sem_i1, sem_r0, sem_r1, sem_r2,
           sem_s0, sem_s1, sem_s2):
    c = lax.axis_index("c")
    s = lax.axis_index("s")
    wid = s * nc + c
    srcg = (srcg0, srcg1)
    dstg = (dstg0, dstg1)
    sem_i = (sem_i0, sem_i1)
    rows = (rows0, rows1, rows2)
    sem_r = (sem_r0, sem_r1, sem_r2)
    sem_s = (sem_s0, sem_s1, sem_s2)
    rz = n_acc // ns
    pltpu.sync_copy(zacc_hbm.at[pl.ds(s * rz, rz)],
                    acc_sh.at[pl.ds(s * rz, rz)])
    plsc.subcore_barrier()

    def start_idx(g, gs):
      return (
          pltpu.async_copy(src_hbm.at[wid, pl.ds(g * g8, g8)], srcg[gs],
                           sem_i[gs]),
          pltpu.async_copy(dst_hbm.at[wid, pl.ds(g * g8, g8)], dstg[gs],
                           sem_i[gs]),
      )

    def start_gather(j, b):
      return pltpu.async_copy(x_hbm.at[srcg[(j // g8) % 2].at[j % g8]],
                              rows[b], sem_r[b])

    def start_scatter(j, b):
      return pltpu.async_copy(rows[b], acc_sh.at[dstg[(j // g8) % 2].at[j % g8]],
                              sem_s[b], add=True)

    hi = [start_idx(0, 0), start_idx(1, 1)]
    for h in hi[0]:
      h.wait()
    hg = [start_gather(0, 0), start_gather(1, 1), None]
    hs = [None, None, None]
    for j in range(nch):
      g, t, gs, b = j // g8, j % g8, (j // g8) % 2, j % nb
      if t == 6 and g + 1 < ng:
        for h in hi[gs ^ 1]:
          h.wait()
      hg[b].wait()
      hs[b] = start_scatter(j, b)
      nxt = j + 2
      if nxt < nch:
        bb = nxt % nb
        if hs[bb] is not None:
          hs[bb].wait()
        hg[bb] = start_gather(nxt, bb)
      if t == 7 and g + 2 < ng:
        hi[gs] = start_idx(g + 2, gs)
    for b in range(nb):
      if hs[b] is not None:
        hs[b].wait()
    plsc.subcore_barrier()
    pltpu.sync_copy(acc_sh.at[pl.ds(s * rz, rz)],
                    s_out.at[c, pl.ds(s * rz, rz)])

  return pl.kernel(
      body, mesh=mesh,
      out_type=jax.ShapeDtypeStruct((nc, n_acc, d), jnp.float32),
      scratch_types=scratch)


def _make_cnt(n, d, nw, nch, k, n_acc):
  info = plsc.get_sparse_core_info()
  nc, ns = info.num_cores, info.num_subcores
  mesh = plsc.VectorSubcoreMesh(core_axis_name="c", subcore_axis_name="s")
  g8 = 8
  ng = nch // g8
  scratch = [
      pltpu.VMEM((g8, k), jnp.int32),
      pltpu.VMEM((g8, k), jnp.int32),
      pltpu.VMEM((k, d), jnp.float32),
      pltpu.VMEM_SHARED((n_acc, d), jnp.float32),
      pltpu.SemaphoreType.DMA,
      pltpu.SemaphoreType.DMA,
      pltpu.SemaphoreType.DMA,
      pltpu.SemaphoreType.DMA,
  ]

  def body(ones_hbm, dst_hbm, zacc_hbm, cnt_out,
           dstg0, dstg1, onesv, acc_sh, sem_i0, sem_i1, sem_s0, sem_s1):
    c = lax.axis_index("c")
    s = lax.axis_index("s")
    wid = s * nc + c
    dstg = (dstg0, dstg1)
    sem_i = (sem_i0, sem_i1)
    sem_s = (sem_s0, sem_s1)
    rz = n_acc // ns
    pltpu.sync_copy(zacc_hbm.at[pl.ds(s * rz, rz)],
                    acc_sh.at[pl.ds(s * rz, rz)])
    pltpu.sync_copy(ones_hbm, onesv)
    plsc.subcore_barrier()

    def start_idx(g, gs):
      return pltpu.async_copy(dst_hbm.at[wid, pl.ds(g * g8, g8)], dstg[gs],
                              sem_i[gs])

    hi = [start_idx(0, 0), start_idx(1, 1)]
    hi[0].wait()
    hs = []
    for g in range(ng):
      gs = g % 2
      if g + 1 < ng:
        hi[gs ^ 1].wait()
      for t in range(g8):
        hs.append(pltpu.async_copy(onesv, acc_sh.at[dstg[gs].at[t]],
                                   sem_s[gs], add=True))
      for h in hs:
        h.wait()
      hs = []
      if g + 2 < ng:
        hi[gs] = start_idx(g + 2, gs)
    plsc.subcore_barrier()
    pltpu.sync_copy(acc_sh.at[pl.ds(s * rz, rz)],
                    cnt_out.at[c, pl.ds(s * rz, rz)])

  return pl.kernel(
      body, mesh=mesh,
      out_type=jax.ShapeDtypeStruct((nc, n_acc, d), jnp.float32),
      scratch_types=scratch)


def _dense(layer, n, dout):
  def body(*refs):
    if layer == 0:
      (s_ref, cnt_ref, x_ref, wl_ref, bl_ref, wr_ref, g_ref, be_ref,
       o_ref, inv_ref) = refs
      cnt = cnt_ref[...]
      inv = 1.0 / jnp.maximum(cnt[0, :n, 0:1] + cnt[1, :n, 0:1], 1.0)
      inv_ref[...] = inv
    elif layer == 4:
      (s_ref, invin_ref, x_ref, wl_ref, bl_ref, wr_ref, g_ref, be_ref,
       cw1_ref, cb1_ref, cw2_ref, cb2_ref, o_ref) = refs
      inv = invin_ref[...]
    else:
      (s_ref, invin_ref, x_ref, wl_ref, bl_ref, wr_ref, g_ref, be_ref,
       o_ref) = refs
      inv = invin_ref[...]
    sp = s_ref[...]
    mean = (sp[0, :n] + sp[1, :n]) * inv
    out = (jnp.dot(mean, wl_ref[...], preferred_element_type=jnp.float32)
           + bl_ref[...]
           + jnp.dot(x_ref[...], wr_ref[...],
                     preferred_element_type=jnp.float32))
    nrm = jnp.sqrt(jnp.sum(out * out, axis=1, keepdims=True))
    out = out / jnp.maximum(nrm, 1e-12)
    mu = jnp.mean(out, axis=0, keepdims=True)
    var = jnp.mean((out - mu) ** 2, axis=0, keepdims=True)
    out = (out - mu) / jnp.sqrt(var + 1e-5) * g_ref[...] + be_ref[...]
    if layer < 4:
      o_ref[...] = jnp.maximum(out, 0.0)
    else:
      h = jnp.maximum(
          jnp.dot(out, cw1_ref[...], preferred_element_type=jnp.float32)
          + cb1_ref[...], 0.0)
      o_ref[...] = (jnp.dot(h, cw2_ref[...],
                            preferred_element_type=jnp.float32)
                    + cb2_ref[...])

  if layer == 0:
    out_shape = (jax.ShapeDtypeStruct((n, dout), jnp.float32),
                 jax.ShapeDtypeStruct((n, 1), jnp.float32))
  elif layer == 4:
    out_shape = jax.ShapeDtypeStruct((n, 1), jnp.float32)
  else:
    out_shape = jax.ShapeDtypeStruct((n, dout), jnp.float32)
  return pl.pallas_call(body, out_shape=out_shape)


def kernel(x, edge_index,
           Wl0, bl0, Wr0, g0, be0,
           Wl1, bl1, Wr1, g1, be1,
           Wl2, bl2, Wr2, g2, be2,
           Wl3, bl3, Wr3, g3, be3,
           Wl4, bl4, Wr4, g4, be4,
           cW1, cb1, cW2, cb2):
  n, d = x.shape
  e = edge_index.shape[1]
  info = plsc.get_sparse_core_info()
  nw = info.num_cores * info.num_subcores
  ns = info.num_subcores
  k = 80
  nch = -(-e // (nw * k * 16)) * 16
  e_pad = nw * nch * k
  pad = e_pad - e
  n_acc = -(-n // (ns * 8)) * ns * 8
  npad_rows = n_acc - n
  src = edge_index[0]
  dst = edge_index[1]
  if pad:
    ar = jnp.arange(pad, dtype=jnp.int32)
    src = jnp.concatenate([src, ar % n])
    dst = jnp.concatenate([dst, n + (ar % max(npad_rows, 1))])
  src_p = src.reshape(nw, nch, k)
  dst_p = dst.reshape(nw, nch, k)
  zacc = jnp.zeros((n_acc, d), jnp.float32)
  ones_rows = jnp.ones((k, d), jnp.float32)

  agg = _make_agg(n, d, nw, nch, k, n_acc)
  cntk = _make_cnt(n, d, nw, nch, k, n_acc)

  layer_params = [
      (Wl0, bl0, Wr0, g0, be0),
      (Wl1, bl1, Wr1, g1, be1),
      (Wl2, bl2, Wr2, g2, be2),
      (Wl3, bl3, Wr3, g3, be3),
      (Wl4, bl4, Wr4, g4, be4),
  ]

  cnt_parts = cntk(ones_rows, dst_p, zacc)
  xi = x
  inv = None
  logits = None
  for i, (wl, bl, wr, g, be) in enumerate(layer_params):
    bl2 = bl.reshape(1, -1)
    g2 = g.reshape(1, -1)
    be2 = be.reshape(1, -1)
    dout = wl.shape[1]
    s_parts = agg(xi, src_p, dst_p, zacc)
    if i == 0:
      xi, inv = _dense(0, n, dout)(s_parts, cnt_parts, xi, wl, bl2, wr,
                                   g2, be2)
    elif i < 4:
      xi = _dense(i, n, dout)(s_parts, inv, xi, wl, bl2, wr, g2, be2)
    else:
      logits = _dense(4, n, dout)(s_parts, inv, xi, wl, bl2, wr, g2, be2,
                                  cW1, cb1.reshape(1, -1),
                                  cW2, cb2.reshape(1, -1))
  return logits[:, 0]
